# baseline (speedup 1.0000x reference)
import jax
import jax.numpy as jnp
from jax.experimental import pallas as pl
from jax.experimental.pallas import tpu as pltpu


_M = 16
_N = 16
_SUB = 2


def _kd_attn_kernel(q_ref, k_ref, v_ref, o_ref, q_acc, k_acc, v_acc):
    h = pl.program_id(1)
    qT = q_ref[0]
    kT = k_ref[0]
    vT = v_ref[0]
    d, S_sub = qT.shape
    n_sub = _M // _SUB
    p = S_sub // n_sub

    def group_sum(xT):
        acc = xT[:, 0:p]
        for g in range(1, n_sub):
            acc = acc + xT[:, g * p:(g + 1) * p]
        return acc

    q_part = group_sum(qT)
    k_part = group_sum(kT)
    v_part = group_sum(vT)

    @pl.when(h == 0)
    def _():
        q_acc[...] = q_part
        k_acc[...] = k_part
        v_acc[...] = v_part

    @pl.when(h == _SUB - 1)
    def _():
        _attend(d, p, q_acc[...] + q_part, k_acc[...] + k_part,
                v_acc[...] + v_part, o_ref)


def _attend(d, p, q_sum, k_sum, v_sum, o_ref):
    q_repT = q_sum * (1.0 / _M)
    k_repT = k_sum * (1.0 / _N)
    v_repT = v_sum * (1.0 / _N)
    scale = d ** -0.5
    wT = jax.lax.dot_general(
        k_repT, q_repT, (((0,), (0,)), ((), ())),
        preferred_element_type=jnp.float32) * scale
    w_max = jnp.max(wT, axis=0, keepdims=True)
    e = jnp.exp(wT - w_max)
    softT = e / jnp.sum(e, axis=0, keepdims=True)
    out_repT = jax.lax.dot_general(
        v_repT, softT, (((1,), (0,)), ((), ())),
        preferred_element_type=jnp.float32)
    for g in range(_M):
        o_ref[0, :, g * p:(g + 1) * p] = out_repT


def kernel(query, key, value, n_query_groups, n_key_groups):
    del n_query_groups, n_key_groups
    B, H, S, d = query.shape
    BH = B * H
    qT = jnp.swapaxes(query, 2, 3).reshape(BH, d, S)
    kT = jnp.swapaxes(key, 2, 3).reshape(BH, d, S)
    vT = jnp.swapaxes(value, 2, 3).reshape(BH, d, S)
    in_spec = pl.BlockSpec((1, d, S // _SUB), lambda i, h: (i, 0, h))
    outT = pl.pallas_call(
        _kd_attn_kernel,
        grid=(BH, _SUB),
        in_specs=[in_spec, in_spec, in_spec],
        out_specs=pl.BlockSpec((1, d, S), lambda i, h: (i, 0, 0)),
        out_shape=jax.ShapeDtypeStruct((BH, d, S), jnp.float32),
        scratch_shapes=[pltpu.VMEM((d, S // _M), jnp.float32)] * 3,
    )(qT, kT, vT)
    return jnp.swapaxes(outT.reshape(B, H, d, S), 2, 3)


# speedup vs baseline: 1.4021x; 1.4021x over previous
import jax
import jax.numpy as jnp
from jax.experimental import pallas as pl


_M = 16
_N = 16


def _kd_attn_kernel(q_ref, k_ref, v_ref, o_ref):
    qT = q_ref[0]
    kT = k_ref[0]
    vT = v_ref[0]
    d, S = qT.shape
    p = S // _M

    def group_mean(xT, n):
        acc = xT[:, 0:p]
        for g in range(1, n):
            acc = acc + xT[:, g * p:(g + 1) * p]
        return acc * (1.0 / n)

    q_repT = group_mean(qT, _M)
    k_repT = group_mean(kT, _N)
    v_repT = group_mean(vT, _N)
    scale = d ** -0.5
    wT = jax.lax.dot_general(
        k_repT, q_repT, (((0,), (0,)), ((), ())),
        preferred_element_type=jnp.float32) * scale
    w_max = jnp.max(wT, axis=0, keepdims=True)
    e = jnp.exp(wT - w_max)
    softT = e / jnp.sum(e, axis=0, keepdims=True)
    out_repT = jax.lax.dot_general(
        v_repT, softT, (((1,), (0,)), ((), ())),
        preferred_element_type=jnp.float32)
    for g in range(_M):
        o_ref[0, :, g * p:(g + 1) * p] = out_repT


def kernel(query, key, value, n_query_groups, n_key_groups):
    del n_query_groups, n_key_groups
    B, H, S, d = query.shape
    BH = B * H
    qT = jnp.swapaxes(query, 2, 3).reshape(BH, d, S)
    kT = jnp.swapaxes(key, 2, 3).reshape(BH, d, S)
    vT = jnp.swapaxes(value, 2, 3).reshape(BH, d, S)
    in_spec = pl.BlockSpec((1, d, S), lambda i: (i, 0, 0))
    outT = pl.pallas_call(
        _kd_attn_kernel,
        grid=(BH,),
        in_specs=[in_spec, in_spec, in_spec],
        out_specs=pl.BlockSpec((1, d, S), lambda i: (i, 0, 0)),
        out_shape=jax.ShapeDtypeStruct((BH, d, S), jnp.float32),
    )(qT, kT, vT)
    return jnp.swapaxes(outT.reshape(B, H, d, S), 2, 3)
